# baseline (speedup 1.0000x reference)
import functools

import jax
import jax.numpy as jnp
from jax import lax
from jax.experimental import pallas as pl
from jax.experimental.pallas import tpu as pltpu
from jax.experimental.pallas import tpu_sc as plsc


@functools.lru_cache(maxsize=None)
def _build_gather(n_tokens: int, hidden: int):
    info = plsc.get_sparse_core_info()
    nc, ns = info.num_cores, info.num_subcores
    nw = nc * ns
    assert n_tokens % nw == 0
    rows_per_w = n_tokens // nw
    chunk = 8
    nbuf = 4
    n_chunks = rows_per_w // chunk

    mesh = plsc.VectorSubcoreMesh(core_axis_name="c", subcore_axis_name="s")

    @functools.partial(
        pl.kernel,
        mesh=mesh,
        out_type=jax.ShapeDtypeStruct((n_tokens, hidden), jnp.float32),
        scratch_types=[
            pltpu.VMEM((rows_per_w,), jnp.int32),
            pltpu.VMEM((nbuf, chunk, hidden), jnp.float32),
        ]
        + [pltpu.SemaphoreType.DMA] * (2 * nbuf),
    )
    def gather_k(table_hbm, idx_hbm, out_hbm, idx_v, bufs, *sems):
        gsems, ssems = sems[:nbuf], sems[nbuf:]
        wid = lax.axis_index("s") * nc + lax.axis_index("c")
        base = wid * rows_per_w
        pltpu.sync_copy(idx_hbm.at[pl.ds(base, rows_per_w)], idx_v)

        def gather_start(i, b):
            pltpu.async_copy(
                table_hbm.at[idx_v.at[pl.ds(i * chunk, chunk)]], bufs.at[b], gsems[b]
            )

        def gather_wait(i, b):
            pltpu.make_async_copy(
                table_hbm.at[idx_v.at[pl.ds(i * chunk, chunk)]], bufs.at[b], gsems[b]
            ).wait()

        def scatter_start(i, b):
            pltpu.async_copy(
                bufs.at[b], out_hbm.at[pl.ds(base + i * chunk, chunk)], ssems[b]
            )

        def scatter_wait(i, b):
            pltpu.make_async_copy(
                bufs.at[b], out_hbm.at[pl.ds(base + i * chunk, chunk)], ssems[b]
            ).wait()

        for b in range(nbuf):
            gather_start(b, b)

        def body(t, carry):
            g = t * nbuf
            for b in range(nbuf):
                gather_wait(g + b, b)
                scatter_start(g + b, b)
            for b in range(nbuf):
                j = g + nbuf + b

                @pl.when(j < n_chunks)
                def _(b=b, j=j):
                    scatter_wait(j - nbuf, b)
                    gather_start(j, b)

            return carry

        lax.fori_loop(0, n_chunks // nbuf, body, 0)

        for b in range(nbuf):
            scatter_wait(n_chunks - nbuf + b, b)

    return gather_k


def kernel(input_ids, embed_weight):
    bsz, seq = input_ids.shape
    vocab, hidden = embed_weight.shape
    ids = input_ids.reshape(-1).astype(jnp.int32)
    flat = _build_gather(bsz * seq, hidden)(embed_weight, ids)
    text_embeds = flat.reshape(bsz, seq, hidden)
    position_ids = jnp.broadcast_to(jnp.arange(seq, dtype=jnp.int32), (bsz, seq))
    lb_loss = jnp.zeros((1,), dtype=text_embeds.dtype)
    return (text_embeds, position_ids, lb_loss)


# speedup vs baseline: 1.0004x; 1.0004x over previous
import functools

import jax
import jax.numpy as jnp
from jax import lax
from jax.experimental import pallas as pl
from jax.experimental.pallas import tpu as pltpu
from jax.experimental.pallas import tpu_sc as plsc


@functools.lru_cache(maxsize=None)
def _build_gather(bsz: int, seq: int, hidden: int):
    info = plsc.get_sparse_core_info()
    nc, ns = info.num_cores, info.num_subcores
    nw = nc * ns
    n_tokens = bsz * seq
    rows_per_w = n_tokens // nw
    w_per_row = seq // rows_per_w
    chunk = 32
    n_chunks = rows_per_w // chunk

    mesh = plsc.VectorSubcoreMesh(core_axis_name="c", subcore_axis_name="s")

    @functools.partial(
        pl.kernel,
        mesh=mesh,
        out_type=jax.ShapeDtypeStruct((n_tokens, hidden), jnp.float32),
        scratch_types=[
            pltpu.VMEM((rows_per_w,), jnp.int32),
            pltpu.VMEM((2, chunk, hidden), jnp.float32),
            pltpu.SemaphoreType.DMA,
            pltpu.SemaphoreType.DMA,
            pltpu.SemaphoreType.DMA,
            pltpu.SemaphoreType.DMA,
        ],
    )
    def gather_k(table_hbm, idx_hbm, out_hbm, idx_v, bufs, g0, g1, s0, s1):
        wid = lax.axis_index("s") * nc + lax.axis_index("c")
        base = wid * rows_per_w
        pltpu.sync_copy(
            idx_hbm.at[wid // w_per_row].at[
                pl.ds(lax.rem(wid, w_per_row) * rows_per_w, rows_per_w)
            ],
            idx_v,
        )

        def gather(i, b, sem):
            return pltpu.make_async_copy(
                table_hbm.at[idx_v.at[pl.ds(i * chunk, chunk)]], bufs.at[b], sem
            )

        def scatter(i, b, sem):
            return pltpu.make_async_copy(
                bufs.at[b], out_hbm.at[pl.ds(base + i * chunk, chunk)], sem
            )

        n_groups = n_chunks // 2
        gather(0, 0, g0).start()

        def body(t, carry):
            i0 = 2 * t
            i1 = i0 + 1
            gather(i0, 0, g0).wait()
            scatter(i0, 0, s0).start()

            @pl.when(t > 0)
            def _():
                scatter(i1 - 2, 1, s1).wait()

            gather(i1, 1, g1).start()
            gather(i1, 1, g1).wait()
            scatter(i1, 1, s1).start()

            @pl.when(t + 1 < n_groups)
            def _():
                scatter(i0, 0, s0).wait()
                gather(i0 + 2, 0, g0).start()

            return carry

        lax.fori_loop(0, n_groups, body, 0)

        scatter(n_chunks - 2, 0, s0).wait()
        scatter(n_chunks - 1, 1, s1).wait()

    return gather_k


def kernel(input_ids, embed_weight):
    bsz, seq = input_ids.shape
    vocab, hidden = embed_weight.shape
    ids = input_ids.astype(jnp.int32)
    flat = _build_gather(bsz, seq, hidden)(embed_weight, ids)
    text_embeds = flat.reshape(bsz, seq, hidden)
    position_ids = jnp.broadcast_to(jnp.arange(seq, dtype=jnp.int32), (bsz, seq))
    lb_loss = jnp.zeros((1,), dtype=text_embeds.dtype)
    return (text_embeds, position_ids, lb_loss)


# speedup vs baseline: 1.0036x; 1.0032x over previous
import functools

import jax
import jax.numpy as jnp
from jax import lax
from jax.experimental import pallas as pl
from jax.experimental.pallas import tpu as pltpu
from jax.experimental.pallas import tpu_sc as plsc


@functools.lru_cache(maxsize=None)
def _build_gather(n_tokens: int, hidden: int):
    info = plsc.get_sparse_core_info()
    nc, ns = info.num_cores, info.num_subcores
    nw = nc * ns
    assert n_tokens % nw == 0
    rows_per_w = n_tokens // nw
    chunk = 8
    nbuf = 4
    n_chunks = rows_per_w // chunk

    mesh = plsc.VectorSubcoreMesh(core_axis_name="c", subcore_axis_name="s")

    @functools.partial(
        pl.kernel,
        mesh=mesh,
        out_type=jax.ShapeDtypeStruct((n_tokens, hidden), jnp.float32),
        scratch_types=[
            pltpu.VMEM((rows_per_w,), jnp.int32),
            pltpu.VMEM((nbuf, chunk, hidden), jnp.float32),
        ]
        + [pltpu.SemaphoreType.DMA] * (2 * nbuf),
    )
    def gather_k(table_hbm, idx_hbm, out_hbm, idx_v, bufs, *sems):
        gsems, ssems = sems[:nbuf], sems[nbuf:]
        wid = lax.axis_index("s") * nc + lax.axis_index("c")
        base = wid * rows_per_w
        head = nbuf * chunk
        head_cp = pltpu.make_async_copy(
            idx_hbm.at[pl.ds(base, head)], idx_v.at[pl.ds(0, head)], gsems[0]
        )
        tail_cp = pltpu.make_async_copy(
            idx_hbm.at[pl.ds(base + head, rows_per_w - head)],
            idx_v.at[pl.ds(head, rows_per_w - head)],
            ssems[0],
        )
        head_cp.start()
        tail_cp.start()
        head_cp.wait()

        def gather_start(i, b):
            pltpu.async_copy(
                table_hbm.at[idx_v.at[pl.ds(i * chunk, chunk)]], bufs.at[b], gsems[b]
            )

        def gather_wait(i, b):
            pltpu.make_async_copy(
                table_hbm.at[idx_v.at[pl.ds(i * chunk, chunk)]], bufs.at[b], gsems[b]
            ).wait()

        def scatter_start(i, b):
            pltpu.async_copy(
                bufs.at[b], out_hbm.at[pl.ds(base + i * chunk, chunk)], ssems[b]
            )

        def scatter_wait(i, b):
            pltpu.make_async_copy(
                bufs.at[b], out_hbm.at[pl.ds(base + i * chunk, chunk)], ssems[b]
            ).wait()

        for b in range(nbuf):
            gather_start(b, b)
        tail_cp.wait()

        def body(t, carry):
            g = t * nbuf
            for b in range(nbuf):
                gather_wait(g + b, b)
                scatter_start(g + b, b)
            for b in range(nbuf):
                j = g + nbuf + b

                @pl.when(j < n_chunks)
                def _(b=b, j=j):
                    scatter_wait(j - nbuf, b)
                    gather_start(j, b)

            return carry

        lax.fori_loop(0, n_chunks // nbuf, body, 0)

        for b in range(nbuf):
            scatter_wait(n_chunks - nbuf + b, b)

    return gather_k


def kernel(input_ids, embed_weight):
    bsz, seq = input_ids.shape
    vocab, hidden = embed_weight.shape
    ids = input_ids.reshape(-1).astype(jnp.int32)
    flat = _build_gather(bsz * seq, hidden)(embed_weight, ids)
    text_embeds = flat.reshape(bsz, seq, hidden)
    position_ids = jnp.broadcast_to(jnp.arange(seq, dtype=jnp.int32), (bsz, seq))
    lb_loss = jnp.zeros((1,), dtype=text_embeds.dtype)
    return (text_embeds, position_ids, lb_loss)


# speedup vs baseline: 1.0244x; 1.0207x over previous
import functools

import jax
import jax.numpy as jnp
from jax import lax
from jax.experimental import pallas as pl
from jax.experimental.pallas import tpu as pltpu
from jax.experimental.pallas import tpu_sc as plsc


@functools.lru_cache(maxsize=None)
def _build_gather(n_tokens: int, hidden: int):
    info = plsc.get_sparse_core_info()
    nc, ns = info.num_cores, info.num_subcores
    nw = nc * ns
    assert n_tokens % nw == 0
    rows_per_w = n_tokens // nw
    chunk = 8
    nbuf = 8
    n_chunks = rows_per_w // chunk

    mesh = plsc.VectorSubcoreMesh(core_axis_name="c", subcore_axis_name="s")

    @functools.partial(
        pl.kernel,
        mesh=mesh,
        out_type=jax.ShapeDtypeStruct((n_tokens, hidden), jnp.float32),
        scratch_types=[
            pltpu.VMEM((rows_per_w,), jnp.int32),
            pltpu.VMEM((nbuf, chunk, hidden), jnp.float32),
        ]
        + [pltpu.SemaphoreType.DMA] * (2 * nbuf),
    )
    def gather_k(table_hbm, idx_hbm, out_hbm, idx_v, bufs, *sems):
        gsems, ssems = sems[:nbuf], sems[nbuf:]
        wid = lax.axis_index("s") * nc + lax.axis_index("c")
        base = wid * rows_per_w
        head = nbuf * chunk
        head_cp = pltpu.make_async_copy(
            idx_hbm.at[pl.ds(base, head)], idx_v.at[pl.ds(0, head)], gsems[0]
        )
        tail_cp = pltpu.make_async_copy(
            idx_hbm.at[pl.ds(base + head, rows_per_w - head)],
            idx_v.at[pl.ds(head, rows_per_w - head)],
            ssems[0],
        )
        head_cp.start()
        tail_cp.start()
        head_cp.wait()

        def gather_start(i, b):
            pltpu.async_copy(
                table_hbm.at[idx_v.at[pl.ds(i * chunk, chunk)]], bufs.at[b], gsems[b]
            )

        def gather_wait(i, b):
            pltpu.make_async_copy(
                table_hbm.at[idx_v.at[pl.ds(i * chunk, chunk)]], bufs.at[b], gsems[b]
            ).wait()

        def scatter_start(i, b):
            pltpu.async_copy(
                bufs.at[b], out_hbm.at[pl.ds(base + i * chunk, chunk)], ssems[b]
            )

        def scatter_wait(i, b):
            pltpu.make_async_copy(
                bufs.at[b], out_hbm.at[pl.ds(base + i * chunk, chunk)], ssems[b]
            ).wait()

        for b in range(nbuf):
            gather_start(b, b)
        tail_cp.wait()

        def body(t, carry):
            g = t * nbuf
            for b in range(nbuf):
                gather_wait(g + b, b)
                scatter_start(g + b, b)
            for b in range(nbuf):
                j = g + nbuf + b

                @pl.when(j < n_chunks)
                def _(b=b, j=j):
                    scatter_wait(j - nbuf, b)
                    gather_start(j, b)

            return carry

        lax.fori_loop(0, n_chunks // nbuf, body, 0)

        for b in range(nbuf):
            scatter_wait(n_chunks - nbuf + b, b)

    return gather_k


def kernel(input_ids, embed_weight):
    bsz, seq = input_ids.shape
    vocab, hidden = embed_weight.shape
    ids = input_ids.reshape(-1).astype(jnp.int32)
    flat = _build_gather(bsz * seq, hidden)(embed_weight, ids)
    text_embeds = flat.reshape(bsz, seq, hidden)
    position_ids = jnp.broadcast_to(jnp.arange(seq, dtype=jnp.int32), (bsz, seq))
    lb_loss = jnp.zeros((1,), dtype=text_embeds.dtype)
    return (text_embeds, position_ids, lb_loss)


# speedup vs baseline: 1.0464x; 1.0216x over previous
import functools

import jax
import jax.numpy as jnp
from jax import lax
from jax.experimental import pallas as pl
from jax.experimental.pallas import tpu as pltpu
from jax.experimental.pallas import tpu_sc as plsc


@functools.lru_cache(maxsize=None)
def _build_gather(n_tokens: int, hidden: int):
    info = plsc.get_sparse_core_info()
    nc, ns = info.num_cores, info.num_subcores
    nw = nc * ns
    assert n_tokens % nw == 0
    rows_per_w = n_tokens // nw
    chunk = 8
    nbuf = 15
    look = 8
    n_chunks = rows_per_w // chunk

    mesh = plsc.VectorSubcoreMesh(core_axis_name="c", subcore_axis_name="s")

    @functools.partial(
        pl.kernel,
        mesh=mesh,
        out_type=jax.ShapeDtypeStruct((n_tokens, hidden), jnp.float32),
        scratch_types=[
            pltpu.VMEM((rows_per_w,), jnp.int32),
            pltpu.VMEM((nbuf, chunk, hidden), jnp.float32),
            pltpu.SemaphoreType.DMA,
            pltpu.SemaphoreType.DMA,
        ],
    )
    def gather_k(table_hbm, idx_hbm, out_hbm, idx_v, bufs, gsem, ssem):
        wid = lax.axis_index("s") * nc + lax.axis_index("c")
        base = wid * rows_per_w
        head = look * chunk
        head_cp = pltpu.make_async_copy(
            idx_hbm.at[pl.ds(base, head)], idx_v.at[pl.ds(0, head)], gsem
        )
        tail_cp = pltpu.make_async_copy(
            idx_hbm.at[pl.ds(base + head, rows_per_w - head)],
            idx_v.at[pl.ds(head, rows_per_w - head)],
            ssem,
        )
        head_cp.start()
        tail_cp.start()
        head_cp.wait()

        def gather_cp(i, b):
            return pltpu.make_async_copy(
                table_hbm.at[idx_v.at[pl.ds(i * chunk, chunk)]], bufs.at[b], gsem
            )

        def scatter_cp(i, b):
            return pltpu.make_async_copy(
                bufs.at[b], out_hbm.at[pl.ds(base + i * chunk, chunk)], ssem
            )

        for j in range(look):
            gather_cp(j, j).start()
        tail_cp.wait()

        def body(i, carry):
            b = lax.rem(i, nbuf)
            gather_cp(i, b).wait()
            scatter_cp(i, b).start()
            k = i + look

            @pl.when(k < n_chunks)
            def _():
                bk = lax.rem(k, nbuf)

                @pl.when(k >= nbuf)
                def _():
                    scatter_cp(k - nbuf, bk).wait()

                gather_cp(k, bk).start()

            return carry

        lax.fori_loop(0, n_chunks, body, 0)

        for i in range(n_chunks - nbuf, n_chunks):
            scatter_cp(i, i % nbuf).wait()

    return gather_k


def kernel(input_ids, embed_weight):
    bsz, seq = input_ids.shape
    vocab, hidden = embed_weight.shape
    ids = input_ids.reshape(-1).astype(jnp.int32)
    flat = _build_gather(bsz * seq, hidden)(embed_weight, ids)
    text_embeds = flat.reshape(bsz, seq, hidden)
    position_ids = jnp.broadcast_to(jnp.arange(seq, dtype=jnp.int32), (bsz, seq))
    lb_loss = jnp.zeros((1,), dtype=text_embeds.dtype)
    return (text_embeds, position_ids, lb_loss)
